# Initial kernel scaffold; baseline (speedup 1.0000x reference)
import jax
import jax.numpy as jnp
from jax.experimental import pallas as pl


def kernel(x, query, pre_mask, pruning_index, ln_g, ln_b, W1, b1, W2, b2, W3, b3, W4, b4, gumbel):
    raise NotImplementedError("write your pallas kernel here")


# speedup vs baseline: 3.9623x; 3.9623x over previous
import jax
import jax.numpy as jnp
from jax.experimental import pallas as pl
from jax.experimental.pallas import tpu as pltpu

_TN = 256


def _gelu(v):
    return 0.5 * v * (1.0 + jax.lax.erf(v * (2.0 ** -0.5)))


def _stage_a(x_ref, pm_ref, lng_ref, lnb_ref, w1_ref, b1_ref, w2l_ref,
             l_ref, g_ref):
    t = pl.program_id(1)
    xv = x_ref[0]
    mu = jnp.mean(xv, axis=1, keepdims=True)
    var = jnp.mean((xv - mu) ** 2, axis=1, keepdims=True)
    vn = (xv - mu) / jnp.sqrt(var + 1e-5) * lng_ref[...] + lnb_ref[...]
    h1 = _gelu(jnp.dot(vn, w1_ref[...], preferred_element_type=jnp.float32)
               + b1_ref[...])
    c_half = h1.shape[1] // 2
    l_ref[0] = jnp.dot(h1[:, :c_half], w2l_ref[...],
                       preferred_element_type=jnp.float32)
    gm = jnp.sum(h1[:, c_half:] * pm_ref[0], axis=0, keepdims=True)

    @pl.when(t == 0)
    def _():
        g_ref[0] = gm

    @pl.when(t != 0)
    def _():
        g_ref[0] = g_ref[0] + gm


def _stage_b(l_ref, gm_ref, q_ref, w2g_ref, w2q_ref, b2_ref, w3_ref, b3_ref,
             w4p_ref, gd_ref, pm_ref, out_ref, qg_ref):
    t = pl.program_id(2)

    @pl.when(t == 0)
    def _():
        g_row = jnp.dot(gm_ref[0], w2g_ref[...],
                        preferred_element_type=jnp.float32)
        q_row = jnp.dot(q_ref[0], w2q_ref[...],
                        preferred_element_type=jnp.float32)
        qg_ref[...] = g_row + q_row + b2_ref[...]

    h2 = _gelu(l_ref[0] + qg_ref[...])
    h3 = _gelu(jnp.dot(h2, w3_ref[...], preferred_element_type=jnp.float32)
               + b3_ref[...])
    logits = jnp.dot(h3, w4p_ref[...],
                     preferred_element_type=jnp.float32)
    delta = logits[:, 0:1] - logits[:, 1:2]
    post = jnp.where(delta + gd_ref[0] >= 0.0, 1.0, 0.0) * pm_ref[0]
    out_ref[0] = post


def kernel(x, query, pre_mask, pruning_index, ln_g, ln_b,
           W1, b1, W2, b2, W3, b3, W4, b4, gumbel):
    N, B, C = x.shape
    n = query.shape[1]
    ch = C // 2

    xt = jnp.transpose(x, (1, 0, 2))
    q = jnp.transpose(query[-1], (1, 0, 2)).reshape(B * n, 1, C)
    w2l, w2g, w2q = W2[:ch], W2[ch:C], W2[C:]
    w4p = jnp.zeros((ch, 128), jnp.float32).at[:, :2].set(W4)
    gd = (gumbel[..., 0] - gumbel[..., 1]
          + (b4[0] - b4[1])).reshape(B * n, N, 1)

    const2 = lambda shape: pl.BlockSpec(shape, lambda b, t: (0, 0))
    L, gsum = pl.pallas_call(
        _stage_a,
        grid=(B, N // _TN),
        in_specs=[
            pl.BlockSpec((1, _TN, C), lambda b, t: (b, t, 0)),
            pl.BlockSpec((1, _TN, 1), lambda b, t: (b, t, 0)),
            const2((1, C)), const2((1, C)),
            const2((C, C)), const2((1, C)),
            const2((ch, C)),
        ],
        out_specs=[
            pl.BlockSpec((1, _TN, C), lambda b, t: (b, t, 0)),
            pl.BlockSpec((1, 1, ch), lambda b, t: (b, 0, 0)),
        ],
        out_shape=[
            jax.ShapeDtypeStruct((B, N, C), jnp.float32),
            jax.ShapeDtypeStruct((B, 1, ch), jnp.float32),
        ],
    )(xt, pre_mask, ln_g.reshape(1, C), ln_b.reshape(1, C),
      W1, b1.reshape(1, C), w2l)

    psum = jnp.sum(pre_mask, axis=1).reshape(B, 1, 1)
    gmean = gsum / psum

    const3 = lambda shape: pl.BlockSpec(shape, lambda b, i, t: (0, 0))
    post = pl.pallas_call(
        _stage_b,
        grid=(B, n, N // _TN),
        in_specs=[
            pl.BlockSpec((1, _TN, C), lambda b, i, t: (b, t, 0)),
            pl.BlockSpec((1, 1, ch), lambda b, i, t: (b, 0, 0)),
            pl.BlockSpec((1, 1, C), lambda b, i, t: (b * n + i, 0, 0)),
            const3((ch, C)), const3((C, C)), const3((1, C)),
            const3((C, ch)), const3((1, ch)),
            const3((ch, 128)),
            pl.BlockSpec((1, _TN, 1), lambda b, i, t: (b * n + i, t, 0)),
            pl.BlockSpec((1, _TN, 1), lambda b, i, t: (b, t, 0)),
        ],
        out_specs=pl.BlockSpec((1, _TN, 1), lambda b, i, t: (b * n + i, t, 0)),
        out_shape=jax.ShapeDtypeStruct((B * n, N, 1), jnp.float32),
        scratch_shapes=[pltpu.VMEM((1, C), jnp.float32)],
    )(L, gmean, q, w2g, w2q, b2.reshape(1, C), W3, b3.reshape(1, ch),
      w4p, gd, pre_mask)

    post_mask = post.reshape(B, n, N, 1)
    loc = jnp.array([2, 3, 4, 5])
    ratio_train = jnp.array([0.6, 0.6, 0.3, 0.3], dtype=jnp.float32)
    gt = ratio_train[jnp.argmax(loc == pruning_index)]
    pred_ratio = jnp.mean(post_mask, axis=2)
    mask_loss = jnp.mean((pred_ratio - gt) ** 2, axis=1)
    return post_mask, mask_loss
